# baseline (speedup 1.0000x reference)
import jax
import jax.numpy as jnp
from jax import lax
from jax.experimental import pallas as pl
from jax.experimental.pallas import tpu as pltpu

_BLOCK = 20000


def _body(h_ref, graph_ref, W_ref, b_ref, out_ref, M_ref, bg_ref):
    @pl.when(pl.program_id(0) == 0)
    def _():
        M_ref[:, :] = lax.dot_general(
            W_ref[:, :], graph_ref[:, :], (((0,), (0,)), ((), ())),
            preferred_element_type=jnp.float32)
        bg_ref[:, :] = jnp.dot(
            b_ref[:, :], graph_ref[:, :], preferred_element_type=jnp.float32)

    base = (pl.program_id(0) % 2) * (_BLOCK // 2)
    out_ref[:, :] = jnp.dot(
        h_ref[pl.ds(base, _BLOCK // 2), :], M_ref[:, :],
        preferred_element_type=jnp.float32) + bg_ref[:, :]


def kernel(h, graph, W, b):
    Bb, T, D = h.shape
    G = graph.shape[1]
    n = Bb * T
    h2 = h.reshape(n, D)
    b2 = b.reshape(1, -1)
    out = pl.pallas_call(
        _body,
        grid=(2 * n // _BLOCK,),
        in_specs=[
            pl.BlockSpec((_BLOCK, D), lambda i: (i // 2, 0)),
            pl.BlockSpec(graph.shape, lambda i: (0, 0)),
            pl.BlockSpec(W.shape, lambda i: (0, 0)),
            pl.BlockSpec((1, G), lambda i: (0, 0)),
        ],
        out_specs=pl.BlockSpec((_BLOCK // 2, G), lambda i: (i, 0)),
        out_shape=jax.ShapeDtypeStruct((n, G), jnp.float32),
        scratch_shapes=[
            pltpu.VMEM((W.shape[1], G), jnp.float32),
            pltpu.VMEM((1, G), jnp.float32),
        ],
        compiler_params=pltpu.CompilerParams(
            dimension_semantics=("arbitrary",)),
    )(h2, graph, W, b2)
    return out.reshape(Bb, T, G)
